# baseline (speedup 1.0000x reference)
import jax
import jax.numpy as jnp
from jax import lax
from jax.experimental import pallas as pl
from jax.experimental.pallas import tpu as pltpu
from jax.experimental.pallas import tpu_sc as plsc

T = 2048
H = 768
I = 1536
E = 8
BM = 256
NPAD = 2 * T + E * BM
NB_R = NPAD // BM
NB_S = T // BM
NCH = 32
CH = (2 * T) // NCH


def _router_body(x_ref, gw_ref, pos1_ref, pos2_ref, w1_ref, w2_ref, be_ref):
    x = x_ref[...]
    gw = gw_ref[...]
    logits = lax.dot_general(gw, x, (((1,), (1,)), ((), ())),
                             preferred_element_type=jnp.float32)
    m = jnp.max(logits, axis=0, keepdims=True)
    ex = jnp.exp(logits - m)
    scores = ex / jnp.sum(ex, axis=0, keepdims=True)

    eio = lax.broadcasted_iota(jnp.int32, (E, T), 0).astype(jnp.float32)
    s1 = jnp.max(scores, axis=0, keepdims=True)
    e1 = jnp.min(jnp.where(scores >= s1, eio, float(E)), axis=0, keepdims=True)
    oh1 = (eio == e1).astype(jnp.float32)
    masked = scores - 2.0 * oh1
    s2 = jnp.max(masked, axis=0, keepdims=True)
    e2 = jnp.min(jnp.where(masked >= s2, eio, float(E)), axis=0, keepdims=True)
    oh2 = (eio == e2).astype(jnp.float32)

    denom = s1 + s2 + 1e-8
    w1_ref[...] = s1 / denom
    w2_ref[...] = s2 / denom

    oh = jnp.concatenate([oh1, oh2], axis=1)
    ohr = oh.reshape(E * NCH, CH)
    uinc = (lax.broadcasted_iota(jnp.int32, (CH, CH), 0) <=
            lax.broadcasted_iota(jnp.int32, (CH, CH), 1)).astype(jnp.float32)
    within = lax.dot_general(ohr, uinc, (((1,), (0,)), ((), ())),
                             preferred_element_type=jnp.float32)
    tot = within[:, CH - 1:CH].reshape(E, NCH)
    uexc = (lax.broadcasted_iota(jnp.int32, (NCH, NCH), 0) <
            lax.broadcasted_iota(jnp.int32, (NCH, NCH), 1)).astype(jnp.float32)
    excl = lax.dot_general(tot, uexc, (((1,), (0,)), ((), ())),
                           preferred_element_type=jnp.float32)
    csum = (within.reshape(E, NCH, CH) + excl[:, :, None]).reshape(E, 2 * T)

    counts = csum[:, 2 * T - 1:2 * T]
    padded = jnp.floor((counts + float(BM - 1)) * (1.0 / BM)) * float(BM)
    linc = (lax.broadcasted_iota(jnp.int32, (E, E), 1) <=
            lax.broadcasted_iota(jnp.int32, (E, E), 0)).astype(jnp.float32)
    cpad = lax.dot_general(linc, padded, (((1,), (0,)), ((), ())),
                           preferred_element_type=jnp.float32)
    offs = cpad - padded

    pos1 = jnp.sum(oh1 * (offs + csum[:, :T] - 1.0), axis=0, keepdims=True)
    pos2 = jnp.sum(oh2 * (offs + csum[:, T:] - 1.0), axis=0, keepdims=True)
    pos1_ref[...] = pos1.astype(jnp.int32)
    pos2_ref[...] = pos2.astype(jnp.int32)

    bio = lax.broadcasted_iota(jnp.int32, (1, NB_R + 1), 1).astype(jnp.float32)
    base = bio * float(BM)
    nfull = jnp.sum((base >= cpad).astype(jnp.float32), axis=0, keepdims=True)
    bef = jnp.minimum(nfull, float(E - 1))
    nact = cpad[E - 1:E, 0:1] * (1.0 / BM)
    bef = jnp.where(bio >= float(NB_R), nact, bef)
    be_ref[...] = bef.astype(jnp.int32)


def _router_call(x, gate_W):
    return pl.pallas_call(
        _router_body,
        out_shape=[
            jax.ShapeDtypeStruct((1, T), jnp.int32),
            jax.ShapeDtypeStruct((1, T), jnp.int32),
            jax.ShapeDtypeStruct((1, T), jnp.float32),
            jax.ShapeDtypeStruct((1, T), jnp.float32),
            jax.ShapeDtypeStruct((1, NB_R + 1), jnp.int32),
        ],
    )(x, gate_W)


def _shared_body(xb_ref, wgs_ref, wus_ref, wds_ref, ysh_ref):
    xs = xb_ref[...]
    hg = lax.dot_general(xs, wgs_ref[0], (((1,), (1,)), ((), ())),
                         preferred_element_type=jnp.float32)
    hu = lax.dot_general(xs, wus_ref[0], (((1,), (1,)), ((), ())),
                         preferred_element_type=jnp.float32)
    act = (hg * (1.0 / (1.0 + jnp.exp(-hg))) * hu).astype(jnp.bfloat16)
    ysh_ref[...] = lax.dot_general(act, wds_ref[0], (((1,), (1,)), ((), ())),
                                   preferred_element_type=jnp.float32)


def _shared_call(x, Wg_sh, Wu_sh, Wd_sh):
    return pl.pallas_call(
        _shared_body,
        grid=(NB_S,),
        in_specs=[
            pl.BlockSpec((BM, H), lambda b: (b, 0)),
            pl.BlockSpec((1, I, H), lambda b: (0, 0, 0)),
            pl.BlockSpec((1, I, H), lambda b: (0, 0, 0)),
            pl.BlockSpec((1, H, I), lambda b: (0, 0, 0)),
        ],
        out_specs=pl.BlockSpec((BM, H), lambda b: (b, 0)),
        out_shape=jax.ShapeDtypeStruct((T, H), jnp.float32),
    )(x.astype(jnp.bfloat16), Wg_sh, Wu_sh, Wd_sh)


_SC_NC = 2
_SC_NS = 16
_NW = _SC_NC * _SC_NS
TPW = T // _NW


def _mlp_body(be_ref, x_ref, pos1_ref, pos2_ref, w1_ref, w2_ref,
              wg_ref, wu_ref, wd_ref, ys_ref):
    b = pl.program_id(0)

    @pl.when(b < be_ref[NB_R])
    def _routed():
        si = lax.broadcasted_iota(jnp.int32, (BM, 1), 0) + b * BM
        m1 = (pos1_ref[...] == si)
        m2 = (pos2_ref[...] == si)
        rw = jnp.sum(jnp.where(m1, w1_ref[...], 0.0) +
                     jnp.where(m2, w2_ref[...], 0.0), axis=1, keepdims=True)
        oh = jnp.where(m1 | m2, 1.0, 0.0).astype(jnp.bfloat16)
        xs = lax.dot_general(oh, x_ref[...], (((1,), (0,)), ((), ())),
                             preferred_element_type=jnp.float32
                             ).astype(jnp.bfloat16)
        hg = lax.dot_general(xs, wg_ref[0], (((1,), (1,)), ((), ())),
                             preferred_element_type=jnp.float32)
        hu = lax.dot_general(xs, wu_ref[0], (((1,), (1,)), ((), ())),
                             preferred_element_type=jnp.float32)
        act = (hg * (1.0 / (1.0 + jnp.exp(-hg))) * hu).astype(jnp.bfloat16)
        out = lax.dot_general(act, wd_ref[0], (((1,), (1,)), ((), ())),
                              preferred_element_type=jnp.float32)
        ys_ref[...] = out * rw


def _mlp_call(be, x, pos1, pos2, w1, w2, Wg, Wu, Wd):
    def _widx(b, be_ref):
        e = jnp.minimum(be_ref[jnp.minimum(b, NB_R - 1)], E - 1)
        return (e, 0, 0)

    grid_spec = pltpu.PrefetchScalarGridSpec(
        num_scalar_prefetch=1,
        grid=(NB_R,),
        in_specs=[
            pl.BlockSpec((T, H), lambda b, be_ref: (0, 0)),
            pl.BlockSpec((1, T), lambda b, be_ref: (0, 0)),
            pl.BlockSpec((1, T), lambda b, be_ref: (0, 0)),
            pl.BlockSpec((1, T), lambda b, be_ref: (0, 0)),
            pl.BlockSpec((1, T), lambda b, be_ref: (0, 0)),
            pl.BlockSpec((1, I, H), _widx),
            pl.BlockSpec((1, I, H), _widx),
            pl.BlockSpec((1, H, I), _widx),
        ],
        out_specs=pl.BlockSpec((BM, H), lambda b, be_ref: (b, 0)),
    )
    return pl.pallas_call(
        _mlp_body,
        grid_spec=grid_spec,
        out_shape=jax.ShapeDtypeStruct((NPAD, H), jnp.float32),
    )(be, x.astype(jnp.bfloat16), pos1, pos2, w1, w2, Wg, Wu, Wd)


def _combine_body(ys_hbm, ysh_hbm, p1_hbm, p2_hbm, out_hbm,
                  idx_v, acc_v, buf_v, sem):
    wid = lax.axis_index("s") * _SC_NC + lax.axis_index("c")
    base = wid * TPW

    pltpu.sync_copy(ysh_hbm.at[pl.ds(base, TPW)], acc_v)

    def _accumulate(p_hbm):
        pltpu.sync_copy(p_hbm.at[pl.ds(base, TPW)], idx_v)
        pltpu.async_copy(ys_hbm.at[idx_v], buf_v, sem).wait()

        def _row(r, carry):
            for c in range(H // 16):
                sl = pl.ds(c * 16, 16)
                acc_v[r, sl] = acc_v[r, sl] + buf_v[r, sl]
            return carry
        lax.fori_loop(0, TPW, _row, 0)

    _accumulate(p1_hbm)
    _accumulate(p2_hbm)
    pltpu.sync_copy(acc_v, out_hbm.at[pl.ds(base, TPW)])


def _combine_call(ys, ysh, pos1, pos2):
    return pl.kernel(
        _combine_body,
        mesh=plsc.VectorSubcoreMesh(core_axis_name="c", subcore_axis_name="s",
                                    num_cores=_SC_NC, num_subcores=_SC_NS),
        out_type=jax.ShapeDtypeStruct((T, H), jnp.float32),
        scratch_types=[
            pltpu.VMEM((TPW,), jnp.int32),
            pltpu.VMEM((TPW, H), jnp.float32),
            pltpu.VMEM((TPW, H), jnp.float32),
            pltpu.SemaphoreType.DMA,
        ],
    )(ys, ysh, pos1, pos2)


def kernel(hidden_states, gate_W, Wg_sh, Wu_sh, Wd_sh, Wg, Wu, Wd):
    b, s, h = hidden_states.shape
    x = hidden_states.reshape(T, H)
    pos1, pos2, w1, w2, be = _router_call(x, gate_W)
    ysh = _shared_call(x, Wg_sh, Wu_sh, Wd_sh)
    p1f = pos1.reshape(T)
    p2f = pos2.reshape(T)
    ys = _mlp_call(be.reshape(NB_R + 1), x, pos1, pos2, w1, w2, Wg, Wu, Wd)
    out = _combine_call(ys, ysh, p1f, p2f)
    return out.reshape(b, s, h)


# speedup vs baseline: 1.0939x; 1.0939x over previous
import jax
import jax.numpy as jnp
from jax import lax
from jax.experimental import pallas as pl
from jax.experimental.pallas import tpu as pltpu
from jax.experimental.pallas import tpu_sc as plsc

T = 2048
H = 768
I = 1536
E = 8
BM = 256
NPAD = 2 * T + E * BM
NB_R = NPAD // BM
NB_S = T // BM
NCH = 32
CH = (2 * T) // NCH


def _router_body(x_ref, gw_ref, pos1_ref, pos2_ref, w1_ref, w2_ref, be_ref):
    x = x_ref[...]
    gw = gw_ref[...]
    logits = lax.dot_general(gw, x, (((1,), (1,)), ((), ())),
                             preferred_element_type=jnp.float32)
    m = jnp.max(logits, axis=0, keepdims=True)
    ex = jnp.exp(logits - m)
    scores = ex / jnp.sum(ex, axis=0, keepdims=True)

    eio = lax.broadcasted_iota(jnp.int32, (E, T), 0).astype(jnp.float32)
    s1 = jnp.max(scores, axis=0, keepdims=True)
    e1 = jnp.min(jnp.where(scores >= s1, eio, float(E)), axis=0, keepdims=True)
    oh1 = (eio == e1).astype(jnp.float32)
    masked = scores - 2.0 * oh1
    s2 = jnp.max(masked, axis=0, keepdims=True)
    e2 = jnp.min(jnp.where(masked >= s2, eio, float(E)), axis=0, keepdims=True)
    oh2 = (eio == e2).astype(jnp.float32)

    denom = s1 + s2 + 1e-8
    w1_ref[...] = s1 / denom
    w2_ref[...] = s2 / denom

    oh = jnp.concatenate([oh1, oh2], axis=1)
    ohr = oh.reshape(E * NCH, CH)
    uinc = (lax.broadcasted_iota(jnp.int32, (CH, CH), 0) <=
            lax.broadcasted_iota(jnp.int32, (CH, CH), 1)).astype(jnp.float32)
    within = lax.dot_general(ohr, uinc, (((1,), (0,)), ((), ())),
                             preferred_element_type=jnp.float32)
    tot = within[:, CH - 1:CH].reshape(E, NCH)
    uexc = (lax.broadcasted_iota(jnp.int32, (NCH, NCH), 0) <
            lax.broadcasted_iota(jnp.int32, (NCH, NCH), 1)).astype(jnp.float32)
    excl = lax.dot_general(tot, uexc, (((1,), (0,)), ((), ())),
                           preferred_element_type=jnp.float32)
    csum = (within.reshape(E, NCH, CH) + excl[:, :, None]).reshape(E, 2 * T)

    counts = csum[:, 2 * T - 1:2 * T]
    padded = jnp.floor((counts + float(BM - 1)) * (1.0 / BM)) * float(BM)
    linc = (lax.broadcasted_iota(jnp.int32, (E, E), 1) <=
            lax.broadcasted_iota(jnp.int32, (E, E), 0)).astype(jnp.float32)
    cpad = lax.dot_general(linc, padded, (((1,), (0,)), ((), ())),
                           preferred_element_type=jnp.float32)
    offs = cpad - padded

    pos1 = jnp.sum(oh1 * (offs + csum[:, :T] - 1.0), axis=0, keepdims=True)
    pos2 = jnp.sum(oh2 * (offs + csum[:, T:] - 1.0), axis=0, keepdims=True)
    pos1_ref[...] = pos1.astype(jnp.int32)
    pos2_ref[...] = pos2.astype(jnp.int32)

    bio = lax.broadcasted_iota(jnp.int32, (1, NB_R + 1), 1).astype(jnp.float32)
    base = bio * float(BM)
    nfull = jnp.sum((base >= cpad).astype(jnp.float32), axis=0, keepdims=True)
    bef = jnp.minimum(nfull, float(E - 1))
    nact = cpad[E - 1:E, 0:1] * (1.0 / BM)
    bef = jnp.where(bio >= float(NB_R), nact, bef)
    be_ref[...] = bef.astype(jnp.int32)


def _router_call(x, gate_W):
    return pl.pallas_call(
        _router_body,
        out_shape=[
            jax.ShapeDtypeStruct((1, T), jnp.int32),
            jax.ShapeDtypeStruct((1, T), jnp.int32),
            jax.ShapeDtypeStruct((1, T), jnp.float32),
            jax.ShapeDtypeStruct((1, T), jnp.float32),
            jax.ShapeDtypeStruct((1, NB_R + 1), jnp.int32),
        ],
    )(x, gate_W)


def _shared_body(xb_ref, wgs_ref, wus_ref, wds_ref, ysh_ref):
    xs = xb_ref[...]
    hg = lax.dot_general(xs, wgs_ref[0], (((1,), (1,)), ((), ())),
                         preferred_element_type=jnp.float32)
    hu = lax.dot_general(xs, wus_ref[0], (((1,), (1,)), ((), ())),
                         preferred_element_type=jnp.float32)
    act = (hg * (1.0 / (1.0 + jnp.exp(-hg))) * hu).astype(jnp.bfloat16)
    ysh_ref[...] = lax.dot_general(act, wds_ref[0], (((1,), (1,)), ((), ())),
                                   preferred_element_type=jnp.float32)


def _shared_call(x, Wg_sh, Wu_sh, Wd_sh):
    return pl.pallas_call(
        _shared_body,
        grid=(NB_S,),
        in_specs=[
            pl.BlockSpec((BM, H), lambda b: (b, 0)),
            pl.BlockSpec((1, I, H), lambda b: (0, 0, 0)),
            pl.BlockSpec((1, I, H), lambda b: (0, 0, 0)),
            pl.BlockSpec((1, H, I), lambda b: (0, 0, 0)),
        ],
        out_specs=pl.BlockSpec((BM, H), lambda b: (b, 0)),
        out_shape=jax.ShapeDtypeStruct((T, H), jnp.float32),
    )(x.astype(jnp.bfloat16), Wg_sh, Wu_sh, Wd_sh)


_SC_NC = 2
_SC_NS = 16
_NW = _SC_NC * _SC_NS
TPW = T // _NW


NB = NB_R + NB_S


def _mlp_body(be_ref, x_ref, pos1_ref, pos2_ref, w1_ref, w2_ref,
              wg_ref, wu_ref, wd_ref, ysh_ref, out_ref, ys_s):
    b = pl.program_id(0)

    @pl.when((b < NB_R) & (b < be_ref[NB_R]))
    def _routed():
        si = lax.broadcasted_iota(jnp.int32, (BM, 1), 0) + b * BM
        m1 = (pos1_ref[...] == si)
        m2 = (pos2_ref[...] == si)
        rw = jnp.sum(jnp.where(m1, w1_ref[...], 0.0) +
                     jnp.where(m2, w2_ref[...], 0.0), axis=1, keepdims=True)
        oh = jnp.where(m1 | m2, 1.0, 0.0).astype(jnp.bfloat16)
        xs = lax.dot_general(oh, x_ref[...], (((1,), (0,)), ((), ())),
                             preferred_element_type=jnp.float32
                             ).astype(jnp.bfloat16)
        hg = lax.dot_general(xs, wg_ref[0], (((1,), (1,)), ((), ())),
                             preferred_element_type=jnp.float32)
        hu = lax.dot_general(xs, wu_ref[0], (((1,), (1,)), ((), ())),
                             preferred_element_type=jnp.float32)
        act = (hg * (1.0 / (1.0 + jnp.exp(-hg))) * hu).astype(jnp.bfloat16)
        out = lax.dot_general(act, wd_ref[0], (((1,), (1,)), ((), ())),
                              preferred_element_type=jnp.float32)
        ys_s[pl.ds(b * BM, BM), :] = (out * rw).astype(jnp.bfloat16)

    @pl.when((b < NB_R) & (b >= be_ref[NB_R]))
    def _inactive():
        ys_s[pl.ds(b * BM, BM), :] = jnp.zeros((BM, H), jnp.bfloat16)

    @pl.when(b >= NB_R)
    def _combine():
        tb = (b - NB_R) * BM
        p1b = pos1_ref[:, pl.ds(tb, BM)].reshape(BM, 1)
        p2b = pos2_ref[:, pl.ds(tb, BM)].reshape(BM, 1)
        sio = lax.broadcasted_iota(jnp.int32, (1, NPAD), 1)
        ohc = jnp.where((p1b == sio) | (p2b == sio),
                        1.0, 0.0).astype(jnp.bfloat16)
        gathered = lax.dot_general(ohc, ys_s[...], (((1,), (0,)), ((), ())),
                                   preferred_element_type=jnp.float32)
        out_ref[...] = gathered + ysh_ref[...]


def _mlp_call(be, x, pos1, pos2, w1, w2, Wg, Wu, Wd, ysh):
    def _widx(b, be_ref):
        e = jnp.minimum(be_ref[jnp.minimum(b, NB_R - 1)], E - 1)
        return (e, 0, 0)

    def _tidx(b, be_ref):
        return (jnp.maximum(b - NB_R, 0), 0)

    grid_spec = pltpu.PrefetchScalarGridSpec(
        num_scalar_prefetch=1,
        grid=(NB,),
        in_specs=[
            pl.BlockSpec((T, H), lambda b, be_ref: (0, 0)),
            pl.BlockSpec((1, T), lambda b, be_ref: (0, 0)),
            pl.BlockSpec((1, T), lambda b, be_ref: (0, 0)),
            pl.BlockSpec((1, T), lambda b, be_ref: (0, 0)),
            pl.BlockSpec((1, T), lambda b, be_ref: (0, 0)),
            pl.BlockSpec((1, I, H), _widx),
            pl.BlockSpec((1, I, H), _widx),
            pl.BlockSpec((1, H, I), _widx),
            pl.BlockSpec((BM, H), _tidx),
        ],
        out_specs=pl.BlockSpec((BM, H), _tidx),
        scratch_shapes=[pltpu.VMEM((NPAD, H), jnp.bfloat16)],
    )
    return pl.pallas_call(
        _mlp_body,
        grid_spec=grid_spec,
        out_shape=jax.ShapeDtypeStruct((T, H), jnp.float32),
    )(be, x.astype(jnp.bfloat16), pos1, pos2, w1, w2, Wg, Wu, Wd, ysh)


def kernel(hidden_states, gate_W, Wg_sh, Wu_sh, Wd_sh, Wg, Wu, Wd):
    b, s, h = hidden_states.shape
    x = hidden_states.reshape(T, H)
    pos1, pos2, w1, w2, be = _router_call(x, gate_W)
    ysh = _shared_call(x, Wg_sh, Wu_sh, Wd_sh)
    out = _mlp_call(be.reshape(NB_R + 1), x, pos1, pos2, w1, w2,
                    Wg, Wu, Wd, ysh)
    return out.reshape(b, s, h)
